# Initial kernel scaffold; baseline (speedup 1.0000x reference)
import jax
import jax.numpy as jnp
from jax.experimental import pallas as pl


def kernel(user_embeddings, item_embeddings, image_embeddings, text_embeddings, user_id, item_id, logit_scale, W_i2t, W_t2i, W_i2d, W_d2i, W_t2d, W_d2t):
    raise NotImplementedError("write your pallas kernel here")


# speedup vs baseline: 2.3111x; 2.3111x over previous
import numpy as np
import jax
import jax.numpy as jnp
from jax.experimental import pallas as pl
from jax.experimental.pallas import tpu as pltpu
from jax.experimental.pallas import tpu_sc as plsc

B = 4096
D = 64
TILE = 256
NT = B // TILE
TC1 = 128
NC1 = B // TC1
NEG = -1e30


def _make_noise_consts():
    key = jax.random.key(42)
    gs, ns = [], []
    for m in range(3):
        km = jax.random.fold_in(key, m)
        u = jax.random.uniform(jax.random.fold_in(km, 1), (B,))
        g = -jnp.log(-jnp.log(u + 1e-20) + 1e-20)
        noise = 0.01 * jax.random.normal(jax.random.fold_in(km, 2), (B, D), jnp.float32)
        gs.append(np.asarray(g, np.float32).reshape(B, 1))
        ns.append(np.asarray(noise, np.float32))
    return gs, ns


_GS, _NOISES = _make_noise_consts()


def _fiota(shape, dim):
    return jax.lax.broadcasted_iota(jnp.int32, shape, dim).astype(jnp.float32)


def _a1_body(iidc, iidr, uidc, uidr, rank_o, cnt_o, first_o, ufirst_o, nuv_o):
    t = pl.program_id(0)
    i0 = t * TILE
    ii = _fiota((TILE, 1), 0) + i0
    jj = _fiota((1, B), 1)
    ltj = jj < ii
    ri = iidc[pl.ds(i0, TILE), :]
    cj = iidr[0:1, :]
    eq = jnp.where(cj == ri, 1.0, 0.0)
    lt = jnp.where(cj < ri, 1.0, 0.0)
    cnt_o[...] = jnp.sum(eq, axis=1, keepdims=True)
    eqb = jnp.sum(jnp.where(ltj, eq, 0.0), axis=1, keepdims=True)
    rank_o[...] = jnp.sum(lt, axis=1, keepdims=True) + eqb
    first_o[...] = jnp.where(eqb == 0.0, 1.0, 0.0)
    ur = uidc[pl.ds(i0, TILE), :]
    cu = uidr[0:1, :]
    ueq = jnp.where(cu == ur, 1.0, 0.0)
    ueqb = jnp.sum(jnp.where(ltj, ueq, 0.0), axis=1, keepdims=True)
    uf = jnp.where(ueqb == 0.0, 1.0, 0.0)
    ufirst_o[...] = uf

    @pl.when(t == 0)
    def _():
        nuv_o[...] = jnp.zeros_like(nuv_o)

    nuv_o[...] = nuv_o[...] + jnp.sum(uf, keepdims=True)


def _a2_body(iidc, iidr, uidc, uidr, first_r, ufirst_r, rdense_o, ru_o):
    t = pl.program_id(0)
    i0 = t * TILE
    ri = iidc[pl.ds(i0, TILE), :]
    lt = jnp.where(iidr[0:1, :] < ri, 1.0, 0.0)
    rdense_o[...] = jnp.sum(first_r[0:1, :] * lt, axis=1, keepdims=True)
    ur = uidc[pl.ds(i0, TILE), :]
    ult = jnp.where(uidr[0:1, :] < ur, 1.0, 0.0)
    ru_o[...] = jnp.sum(ufirst_r[0:1, :] * ult, axis=1, keepdims=True)


def _a3_body(rank_r, first_r, rdense_r, cnt_r, iidr, uidr,
             iids_o, uids_o, rsort_o, cnts_o, uniqi_o, cntu_o):
    t = pl.program_id(0)
    k0 = t * TILE
    kcol = _fiota((TILE, 1), 0) + k0
    sel = jnp.where(rank_r[0:1, :] == kcol, 1.0, 0.0)
    iids_o[...] = jnp.sum(sel * iidr[0:1, :], axis=1, keepdims=True)
    uids_o[...] = jnp.sum(sel * uidr[0:1, :], axis=1, keepdims=True)
    rsort_o[...] = jnp.sum(sel * rdense_r[0:1, :], axis=1, keepdims=True)
    cnts_o[...] = jnp.sum(sel * cnt_r[0:1, :], axis=1, keepdims=True)
    selI = first_r[0:1, :] * jnp.where(rdense_r[0:1, :] == kcol, 1.0, 0.0)
    uniqi_o[...] = jnp.sum(selI * iidr[0:1, :], axis=1, keepdims=True)
    cntu_o[...] = jnp.sum(selI * cnt_r[0:1, :], axis=1, keepdims=True)


def _a4_body(ru_c, rs_c, ru_r, rs_r, d_o):
    t = pl.program_id(0)
    k0 = t * TILE
    kk = _fiota((TILE, 1), 0) + k0
    jj = _fiota((1, B), 1)
    pk_k = ru_c[pl.ds(k0, TILE), :] * float(B) + rs_c[pl.ds(k0, TILE), :]
    pk_j = ru_r[0:1, :] * float(B) + rs_r[0:1, :]
    dup = jnp.sum(jnp.where((pk_j == pk_k) & (jj < kk), 1.0, 0.0),
                  axis=1, keepdims=True)
    d_o[...] = jnp.where(dup == 0.0, 1.0, 0.0)


def _col_spec():
    return pl.BlockSpec((B, 1), lambda t: (0, 0))


def _row_spec():
    return pl.BlockSpec((1, B), lambda t: (0, 0))


def _out_col():
    return pl.BlockSpec((TILE, 1), lambda t: (t, 0))


def _colsd():
    return jax.ShapeDtypeStruct((B, 1), jnp.float32)


def _run_a1(iidc, iidr, uidc, uidr):
    return pl.pallas_call(
        _a1_body,
        grid=(NT,),
        in_specs=[_col_spec(), _row_spec(), _col_spec(), _row_spec()],
        out_specs=[_out_col(), _out_col(), _out_col(), _out_col(),
                   pl.BlockSpec((1, 1), lambda t: (0, 0))],
        out_shape=[_colsd(), _colsd(), _colsd(), _colsd(),
                   jax.ShapeDtypeStruct((1, 1), jnp.float32)],
    )(iidc, iidr, uidc, uidr)


def _run_a2(iidc, iidr, uidc, uidr, first_r, ufirst_r):
    return pl.pallas_call(
        _a2_body,
        grid=(NT,),
        in_specs=[_col_spec(), _row_spec(), _col_spec(), _row_spec(),
                  _row_spec(), _row_spec()],
        out_specs=[_out_col(), _out_col()],
        out_shape=[_colsd(), _colsd()],
    )(iidc, iidr, uidc, uidr, first_r, ufirst_r)


def _run_a3(rank_r, first_r, rdense_r, cnt_r, iidr, uidr):
    return pl.pallas_call(
        _a3_body,
        grid=(NT,),
        in_specs=[_row_spec()] * 6,
        out_specs=[_out_col()] * 6,
        out_shape=[_colsd()] * 6,
    )(rank_r, first_r, rdense_r, cnt_r, iidr, uidr)


def _run_a4(ru_c, rs_c, ru_r, rs_r):
    return pl.pallas_call(
        _a4_body,
        grid=(NT,),
        in_specs=[_col_spec(), _col_spec(), _row_spec(), _row_spec()],
        out_specs=[_out_col()],
        out_shape=[_colsd()],
    )(ru_c, rs_c, ru_r, rs_r)


def _sc_gather(table, idx_2d):
    m = idx_2d.shape[1]
    window = 128
    width = table.shape[1]
    mesh = plsc.VectorSubcoreMesh(core_axis_name="c", subcore_axis_name="s")

    @pl.kernel(out_type=jax.ShapeDtypeStruct((m, width), table.dtype), mesh=mesh)
    def k(x_hbm, i_hbm, o_hbm):
        def body(i_vmem, o_vmem):
            pltpu.sync_copy(x_hbm.at[i_vmem.at[0]], o_vmem)

        pltpu.emit_pipeline(
            body,
            grid=(m // window,),
            in_specs=[pl.BlockSpec((1, window), index_map=lambda i: (0, i))],
            out_specs=[pl.BlockSpec((window, width), index_map=lambda i: (i, 0))],
            core_axis_name=("c", "s"),
            dimension_semantics=(pltpu.PARALLEL,),
        )(i_hbm, o_hbm)

    return k(table, idx_2d)


def _nrm(x, eps=1e-12):
    n = jnp.sqrt(jnp.sum(x * x, axis=1, keepdims=True))
    return x / jnp.maximum(n, eps)


def _half(x, par):
    return jnp.where(par > 0.0, x[:, D:2 * D], x[:, 0:D])


def _c1_body(u_rows, u_sel, it_s, im_s, tx_s, aux, n0, n1, n2,
             ru_r, rs_r, nuv, masks_o,
             uns_scr, sc_scr, sr0_scr, sr1_scr, sr2_scr,
             S_scr, vals2_scr, S2_scr):
    emb_in = [it_s, im_s, tx_s]
    uns_scr[...] = _nrm(_half(u_sel[...], aux[:, 1:2]))
    dc = aux[:, 6:7]

    svals = []
    for m in range(3):
        ne = _nrm(_half(emb_in[m][...], aux[:, 2:3]))
        svals.append(jnp.sum(_nrm(_half(u_rows[...], aux[:, 0:1])) * ne,
                             axis=1, keepdims=True))
        sim = jnp.sum(uns_scr[...] * ne, axis=1, keepdims=True)
        e = jnp.exp(sim)
        p = e / jnp.sum(e)
        sc_scr[:, m:m + 1] = jnp.log(p + 1e-20) + aux[:, 3 + m:4 + m]
    vals = jnp.concatenate(
        [dc * svals[0], dc * svals[0] * svals[0],
         dc * svals[1], dc * svals[1] * svals[1],
         dc * svals[2], dc * svals[2] * svals[2],
         dc, jnp.zeros_like(dc)], axis=1)

    rur = ru_r[0:1, :]
    rsr = rs_r[0:1, :]

    def pu_step(t, acc):
        u0 = t * TC1
        ucol = _fiota((TC1, 1), 0) + u0
        put = jnp.where(rur == ucol, 1.0, 0.0)
        S_scr[pl.ds(u0, TC1), :] = jnp.dot(put, vals,
                                           preferred_element_type=jnp.float32)
        return acc

    jax.lax.fori_loop(0, NC1, pu_step, 0)
    S = S_scr[...]
    r3 = S[:, 6:7]
    den = jnp.sqrt(r3)
    nv = nuv[...]

    def cosrow(r1, r2):
        dd = jnp.maximum(jnp.sqrt(r2) * den, 1e-8)
        return jnp.sum(r1 / dd, keepdims=True) / nv

    sc0 = cosrow(S[:, 0:1], S[:, 1:2])
    sc1 = cosrow(S[:, 2:3], S[:, 3:4])
    sc2 = cosrow(S[:, 4:5], S[:, 5:6])
    mx = jnp.maximum(sc0, jnp.maximum(sc1, sc2))
    e0, e1, e2 = jnp.exp(sc0 - mx), jnp.exp(sc1 - mx), jnp.exp(sc2 - mx)
    tot = e0 + e1 + e2
    nums = []
    for e in (e0, e1, e2):
        ms = e / tot - 1.0 / 3.0
        rate = jnp.maximum(1.0 / (1.0 + jnp.exp(-100.0 * ms)) - 0.5, 0.001)
        nums.append(jnp.floor(float(B) * rate))

    sr_scrs = [sr0_scr, sr1_scr, sr2_scr]

    def t_step(t, acc):
        i0 = t * TC1
        jcol = _fiota((B, 1), 0)
        irow = _fiota((1, TC1), 1) + i0
        msk = jnp.where(jcol == irow, 1.0, 0.0)
        for m in range(3):
            sr_scrs[m][0:1, pl.ds(i0, TC1)] = jnp.sum(
                msk * sc_scr[:, m:m + 1], axis=0, keepdims=True)
        return acc

    jax.lax.fori_loop(0, NC1, t_step, 0)

    n_refs = [n0, n1, n2]

    def mask_step(t, acc):
        k0 = t * TC1
        kk = _fiota((TC1, 1), 0) + k0
        jj = _fiota((1, B), 1)
        par_t = aux[pl.ds(k0, TC1), 2:3]
        cnt_t = aux[pl.ds(k0, TC1), 7:8]
        un_t = uns_scr[pl.ds(k0, TC1), :]
        cols = []
        for m in range(3):
            sk = sc_scr[pl.ds(k0, TC1), m:m + 1]
            sj = sr_scrs[m][0:1, :]
            rnk = (jnp.sum(jnp.where(sj > sk, 1.0, 0.0), axis=1, keepdims=True)
                   + jnp.sum(jnp.where((sj == sk) & (jj < kk), 1.0, 0.0),
                             axis=1, keepdims=True))
            mask = jnp.where(rnk < nums[m], 1.0, 0.0)
            emb_t = _half(emb_in[m][pl.ds(k0, TC1), :], par_t)
            noise_t = n_refs[m][pl.ds(k0, TC1), :]
            noised = emb_t + mask * noise_t
            val = jnp.sum(un_t * _nrm(noised), axis=1, keepdims=True) / cnt_t
            cols.append(val)
        vals2_scr[pl.ds(k0, TC1), :] = jnp.concatenate(
            cols + [jnp.zeros((TC1, 5), jnp.float32)], axis=1)
        return acc

    jax.lax.fori_loop(0, NC1, mask_step, 0)

    def pi_step(t, acc):
        i0 = t * TC1
        icol = _fiota((TC1, 1), 0) + i0
        pit = jnp.where(rsr == icol, 1.0, 0.0)
        S2_scr[pl.ds(i0, TC1), :] = jnp.dot(pit, vals2_scr[...],
                                            preferred_element_type=jnp.float32)
        return acc

    jax.lax.fori_loop(0, NC1, pi_step, 0)
    S2 = S2_scr[...]
    uid_s, uii_s, uit_s = S2[:, 0:1], S2[:, 1:2], S2[:, 2:3]
    i_d = jnp.where(uid_s > uii_s, 1.0, 0.0)
    d_i = jnp.where(uii_s > uid_s, 1.0, 0.0)
    d_t = jnp.where(uit_s > uid_s, 1.0, 0.0)
    t_d = jnp.where(uid_s > uit_s, 1.0, 0.0)
    i_t = jnp.where(uit_s > uii_s, 1.0, 0.0)
    t_i = jnp.where(uii_s > uit_s, 1.0, 0.0)
    masks_o[...] = jnp.concatenate(
        [i_d, d_i, d_t, t_d, i_t, t_i,
         jnp.zeros((B, 2), jnp.float32)], axis=1)


def _run_c1(u_rows, u_sel, it_s, im_s, tx_s, par_uo, par_us, par_is,
            ru_r, rs_r, d_c, cnt_c, nuv):
    aux = jnp.concatenate(
        [par_uo, par_us, par_is,
         jnp.asarray(_GS[0]), jnp.asarray(_GS[1]), jnp.asarray(_GS[2]),
         d_c, cnt_c], axis=1)
    args = [u_rows, u_sel, it_s, im_s, tx_s, aux,
            jnp.asarray(_NOISES[0]), jnp.asarray(_NOISES[1]),
            jnp.asarray(_NOISES[2]), ru_r, rs_r, nuv]
    return pl.pallas_call(
        _c1_body,
        out_shape=jax.ShapeDtypeStruct((B, 8), jnp.float32),
        scratch_shapes=[
            pltpu.VMEM((B, D), jnp.float32),
            pltpu.VMEM((B, 8), jnp.float32),
            pltpu.VMEM((1, B), jnp.float32),
            pltpu.VMEM((1, B), jnp.float32),
            pltpu.VMEM((1, B), jnp.float32),
            pltpu.VMEM((B, 8), jnp.float32),
            pltpu.VMEM((B, 8), jnp.float32),
            pltpu.VMEM((B, 8), jnp.float32),
        ],
    )(*args)


def _c2_body(item_seq, image_seq, text_seq, par_u, masks, cntu_c, cntu_r,
             ls_ref, w_i2t, w_t2i, w_i2d, w_d2i, w_t2d, w_d2t, out,
             p0, p1, p2, p3, p4, p5, b0, b1, b2):
    def nrm0(x):
        return x / jnp.sqrt(jnp.sum(x * x, axis=1, keepdims=True))

    pu = par_u[...]
    b0[...] = nrm0(_half(item_seq[...], pu))
    b1[...] = nrm0(_half(image_seq[...], pu))
    b2[...] = nrm0(_half(text_seq[...], pu))
    cf_n = b0[...]
    img_n = b1[...]
    txt_n = b2[...]
    valid_r = jnp.where(cntu_r[0:1, :] > 0.0, 1.0, 0.0)
    valid_c = jnp.where(cntu_c[...] > 0.0, 1.0, 0.0)
    ls = jnp.exp(ls_ref[...])

    def proj(x, w):
        return jax.lax.dot_general(x, w[...], (((1,), (1,)), ((), ())),
                                   preferred_element_type=jnp.float32) + x

    p0[...] = proj(img_n, w_i2d)
    p1[...] = proj(cf_n, w_d2i)
    p2[...] = proj(cf_n, w_d2t)
    p3[...] = proj(txt_n, w_t2d)
    p4[...] = proj(img_n, w_i2t)
    p5[...] = proj(txt_n, w_t2i)

    pairs = [(p0, cf_n, b0, 0), (p1, img_n, b1, 1),
             (p2, txt_n, b2, 2), (p3, cf_n, b0, 3),
             (p4, txt_n, b2, 4), (p5, img_n, b1, 5)]
    total = jnp.zeros((1, 1), jnp.float32)
    for a_mat, b_mat, b_ref, mi in pairs:

        def ce_step(t, acc, a_mat=a_mat, b_mat=b_mat, b_ref=b_ref, mi=mi):
            r0 = t * TILE
            a_t = a_mat[pl.ds(r0, TILE), :]
            logits = ls * jax.lax.dot_general(
                a_t, b_mat, (((1,), (1,)), ((), ())),
                preferred_element_type=jnp.float32)
            masked = jnp.where(valid_r > 0.0, logits, NEG)
            rmx = jnp.max(masked, axis=1, keepdims=True)
            se = jnp.sum(jnp.where(valid_r > 0.0, jnp.exp(masked - rmx), 0.0),
                         axis=1, keepdims=True)
            lse = rmx + jnp.log(se)
            b_t = b_ref[pl.ds(r0, TILE), :]
            diag = ls * jnp.sum(a_t * b_t, axis=1, keepdims=True)
            v_t = jnp.where(cntu_c[pl.ds(r0, TILE), :] > 0.0, 1.0, 0.0)
            ce = jnp.where(v_t > 0.0, lse - diag, 0.0)
            m_t = masks[pl.ds(r0, TILE), mi:mi + 1]
            return acc + jnp.sum(ce * m_t, keepdims=True)

        s = jax.lax.fori_loop(0, NT, ce_step, jnp.zeros((1, 1), jnp.float32))
        c = jnp.sum(masks[:, mi:mi + 1], keepdims=True)
        total += jnp.where(c > 0.0, s / jnp.maximum(c, 1.0), 0.0)
    out[...] = total


def _run_c2(item_seq, image_seq, text_seq, par_u, masks, cntu_c, cntu_r, ls,
            w_i2t, w_t2i, w_i2d, w_d2i, w_t2d, w_d2t):
    return pl.pallas_call(
        _c2_body,
        out_shape=jax.ShapeDtypeStruct((1, 1), jnp.float32),
        scratch_shapes=[pltpu.VMEM((B, D), jnp.float32)] * 9,
    )(item_seq, image_seq, text_seq, par_u, masks, cntu_c, cntu_r, ls,
      w_i2t, w_t2i, w_i2d, w_d2i, w_t2d, w_d2t)


def kernel(user_embeddings, item_embeddings, image_embeddings, text_embeddings,
           user_id, item_id, logit_scale, W_i2t, W_t2i, W_i2d, W_d2i, W_t2d,
           W_d2t):
    iid_f = item_id.astype(jnp.float32)
    uid_f = user_id.astype(jnp.float32)
    iidc = iid_f.reshape(B, 1)
    iidr = iid_f.reshape(1, B)
    uidc = uid_f.reshape(B, 1)
    uidr = uid_f.reshape(1, B)

    rank, cnt, first, ufirst, nuv = _run_a1(iidc, iidr, uidc, uidr)
    rdense, ru = _run_a2(iidc, iidr, uidc, uidr,
                         first.reshape(1, B), ufirst.reshape(1, B))
    iids, uids, rsort, cnts, uniqi, cntu = _run_a3(
        rank.reshape(1, B), first.reshape(1, B), rdense.reshape(1, B),
        cnt.reshape(1, B), iidr, uidr)
    d, = _run_a4(ru, rsort, ru.reshape(1, B), rsort.reshape(1, B))

    uidx = jnp.concatenate(
        [user_id.reshape(-1), uids.reshape(-1).astype(jnp.int32)]).reshape(1, 2 * B)
    iidx = jnp.concatenate(
        [iids.reshape(-1).astype(jnp.int32),
         uniqi.reshape(-1).astype(jnp.int32)]).reshape(1, 2 * B)
    nu = user_embeddings.shape[0]
    ni = item_embeddings.shape[0]
    ug = _sc_gather(user_embeddings.reshape(nu // 2, 2 * D), uidx // 2)
    ig = _sc_gather(item_embeddings.reshape(ni // 2, 2 * D), iidx // 2)
    mg = _sc_gather(image_embeddings.reshape(ni // 2, 2 * D), iidx // 2)
    tg = _sc_gather(text_embeddings.reshape(ni // 2, 2 * D), iidx // 2)
    upar = (uidx % 2).astype(jnp.float32).reshape(2 * B, 1)
    ipar = (iidx % 2).astype(jnp.float32).reshape(2 * B, 1)
    u_rows, u_sel = ug[:B], ug[B:]
    it_s, item_seq = ig[:B], ig[B:]
    im_s, image_seq = mg[:B], mg[B:]
    tx_s, text_seq = tg[:B], tg[B:]

    masks = _run_c1(u_rows, u_sel, it_s, im_s, tx_s,
                    upar[:B], upar[B:], ipar[:B],
                    ru.reshape(1, B), rsort.reshape(1, B), d, cnts, nuv)
    loss = _run_c2(item_seq, image_seq, text_seq, ipar[B:], masks,
                   cntu, cntu.reshape(1, B), logit_scale.reshape(1, 1),
                   W_i2t, W_t2i, W_i2d, W_d2i, W_t2d, W_d2t)
    return loss[0, 0]
